# baseline (speedup 1.0000x reference)
import jax
import jax.numpy as jnp
from jax import lax
from jax.experimental import pallas as pl
from jax.experimental.pallas import tpu as pltpu

N = 262144
C = 170
_BR = 1024
_NB = N // _BR
_CB = 128
_NCB = 2


def _tc_body(tgt_ref, logits_ref, out_ref, acc_ref):
    i = pl.program_id(0)
    j = pl.program_id(1)

    @pl.when((i == 0) & (j == 0))
    def _():
        acc_ref[...] = jnp.zeros_like(acc_ref)

    x = logits_ref[...]
    t = tgt_ref[0, 0, :]
    cols = j * _CB + lax.broadcasted_iota(jnp.int32, (_BR, _CB), 1)
    d = 1.0 - x
    contrib = jnp.where(cols == t[:, None], d * d, 0.0)
    acc_ref[...] += jnp.sum(contrib.reshape(_BR // 8, 8, _CB), axis=0)

    @pl.when((i == _NB - 1) & (j == _NCB - 1))
    def _():
        out_ref[0, 0] = jnp.sum(acc_ref[...])


@jax.jit
def kernel(contrast_logits, contrast_target):
    tgt = contrast_target.astype(jnp.int32).reshape(_NB, 1, _BR)
    total = pl.pallas_call(
        _tc_body,
        grid=(_NB, _NCB),
        in_specs=[
            pl.BlockSpec((1, 1, _BR), lambda i, j: (i, 0, 0)),
            pl.BlockSpec((_BR, _CB), lambda i, j: (i, j)),
        ],
        out_specs=pl.BlockSpec((1, 1), lambda i, j: (0, 0),
                               memory_space=pltpu.SMEM),
        out_shape=jax.ShapeDtypeStruct((1, 1), jnp.float32),
        scratch_shapes=[pltpu.VMEM((8, _CB), jnp.float32)],
        compiler_params=pltpu.CompilerParams(
            dimension_semantics=("arbitrary", "arbitrary"),
        ),
    )(tgt, contrast_logits)
    return total[0, 0] / N


# speedup vs baseline: 1.8904x; 1.8904x over previous
import jax
import jax.numpy as jnp
from jax import lax
from jax.experimental import pallas as pl
from jax.experimental.pallas import tpu as pltpu

N = 262144
C = 170
_NS_ = 4
_BR = 2048
_SPAN = _NS_ * _BR
_NB = N // _SPAN


def _tc_body(tgt_ref, *refs):
    logit_refs = refs[:_NS_]
    out_ref = refs[_NS_]
    s1_ref, s2_ref = refs[_NS_ + 1], refs[_NS_ + 2]
    i = pl.program_id(0)

    @pl.when(i == 0)
    def _():
        s1_ref[...] = jnp.zeros_like(s1_ref)
        s2_ref[...] = jnp.zeros_like(s2_ref)

    cols = lax.broadcasted_iota(jnp.int32, (_BR, C), 1)
    s1 = jnp.zeros((8, C), jnp.float32)
    s2 = jnp.zeros((8, C), jnp.float32)
    for k in range(_NS_):
        x = logit_refs[k][...]
        t = tgt_ref[0, k, :]
        sx = jnp.where(cols == t[:, None], x, 0.0)
        s1 = s1 + jnp.sum(sx.reshape(_BR // 8, 8, C), axis=0)
        s2 = s2 + jnp.sum((sx * x).reshape(_BR // 8, 8, C), axis=0)
    s1_ref[...] += s1
    s2_ref[...] += s2

    @pl.when(i == _NB - 1)
    def _():
        out_ref[0, 0] = N - 2.0 * jnp.sum(s1_ref[...]) + jnp.sum(s2_ref[...])


@jax.jit
def kernel(contrast_logits, contrast_target):
    tgt = contrast_target.astype(jnp.int32).reshape(_NB, _NS_, _BR)

    def lspec(k):
        return pl.BlockSpec((_BR, C), lambda i, k=k: (i * _NS_ + k, 0))

    total = pl.pallas_call(
        _tc_body,
        grid=(_NB,),
        in_specs=[pl.BlockSpec((1, _NS_, _BR), lambda i: (i, 0, 0))]
        + [lspec(k) for k in range(_NS_)],
        out_specs=pl.BlockSpec((1, 1), lambda i: (0, 0),
                               memory_space=pltpu.SMEM),
        out_shape=jax.ShapeDtypeStruct((1, 1), jnp.float32),
        scratch_shapes=[
            pltpu.VMEM((8, C), jnp.float32),
            pltpu.VMEM((8, C), jnp.float32),
        ],
        compiler_params=pltpu.CompilerParams(
            dimension_semantics=("arbitrary",),
        ),
    )(tgt, *([contrast_logits] * _NS_))
    return total[0, 0] / N
